# speedup vs baseline: 11.4918x; 11.4918x over previous
import functools

import jax
import jax.numpy as jnp
from jax import lax
from jax.experimental import pallas as pl
from jax.experimental.pallas import tpu as pltpu
from jax.experimental.pallas import tpu_sc as plsc

NC = 2
NS = 16
CH = 128

_mesh = functools.partial(
    plsc.VectorSubcoreMesh,
    core_axis_name="c", subcore_axis_name="s", num_cores=NC, num_subcores=NS,
)


def _sc_degree(cols_r, ones16, zeros16, k, acc_rows):
  rpt = acc_rows // NS

  @functools.partial(
      pl.kernel,
      out_type=jax.ShapeDtypeStruct((NC, acc_rows, 16), jnp.float32),
      mesh=_mesh(),
      scratch_types=[
          pltpu.VMEM((k, CH), jnp.int32),
          pltpu.VMEM((CH, 16), jnp.float32),
          pltpu.VMEM_SHARED((acc_rows, 16), jnp.float32),
      ],
  )
  def deg_kernel(cols_hbm, ones_hbm, zeros_hbm, out_hbm, colv, onev, deg_sh):
    c = lax.axis_index("c")
    s = lax.axis_index("s")
    pltpu.sync_copy(zeros_hbm.at[pl.ds(s * rpt, rpt)],
                    deg_sh.at[pl.ds(s * rpt, rpt)])
    pltpu.sync_copy(ones_hbm, onev)
    pltpu.sync_copy(cols_hbm.at[c, s], colv)
    plsc.subcore_barrier()

    @pl.loop(0, k)
    def _(j):
      pltpu.sync_copy(onev, deg_sh.at[colv.at[j]], add=True)

    plsc.subcore_barrier()
    pltpu.sync_copy(deg_sh.at[pl.ds(s * rpt, rpt)],
                    out_hbm.at[c].at[pl.ds(s * rpt, rpt)])

  return deg_kernel(cols_r, ones16, zeros16)


def _sc_aggregate(g, rows_r, cols_r, zeros128, k, acc_rows):
  rpt = acc_rows // NS
  d = g.shape[1]

  @functools.partial(
      pl.kernel,
      out_type=jax.ShapeDtypeStruct((NC, acc_rows, d), jnp.float32),
      mesh=_mesh(),
      scratch_types=[
          pltpu.VMEM((k, CH), jnp.int32),
          pltpu.VMEM((k, CH), jnp.int32),
          pltpu.VMEM((CH, d), jnp.float32),
          pltpu.VMEM_SHARED((acc_rows, d), jnp.float32),
          pltpu.SemaphoreType.DMA,
      ],
  )
  def agg_kernel(g_hbm, rows_hbm, cols_hbm, zeros_hbm, out_hbm,
                 rowv, colv, msg, tmp_sh, sem):
    c = lax.axis_index("c")
    s = lax.axis_index("s")
    pltpu.sync_copy(zeros_hbm.at[pl.ds(s * rpt, rpt)],
                    tmp_sh.at[pl.ds(s * rpt, rpt)])
    pltpu.sync_copy(rows_hbm.at[c, s], rowv)
    pltpu.sync_copy(cols_hbm.at[c, s], colv)
    plsc.subcore_barrier()

    @pl.loop(0, k)
    def _(j):
      pltpu.async_copy(g_hbm.at[rowv.at[j]], msg, sem).wait()
      pltpu.sync_copy(msg, tmp_sh.at[colv.at[j]], add=True)

    plsc.subcore_barrier()
    pltpu.sync_copy(tmp_sh.at[pl.ds(s * rpt, rpt)],
                    out_hbm.at[c].at[pl.ds(s * rpt, rpt)])

  return agg_kernel(g, rows_r, cols_r, zeros128)


def _dot(a, w):
  return jnp.dot(a, w, preferred_element_type=jnp.float32,
                 precision=lax.Precision.HIGHEST)


def _tc_mm(x, w, bn):
  n, d = x.shape

  def body(x_ref, w_ref, o_ref):
    o_ref[...] = _dot(x_ref[...], w_ref[...])

  return pl.pallas_call(
      body,
      grid=(n // bn,),
      in_specs=[
          pl.BlockSpec((bn, d), lambda i: (i, 0)),
          pl.BlockSpec((d, d), lambda i: (0, 0)),
      ],
      out_specs=pl.BlockSpec((bn, d), lambda i: (i, 0)),
      out_shape=jax.ShapeDtypeStruct((n, d), jnp.float32),
  )(x, w)


def _tc_scale0(deg16, p1, bn):
  n, d = p1.shape

  def body(deg_ref, p_ref, dis_ref, g_ref):
    counts = jnp.sum(deg_ref[...], axis=2, keepdims=True)
    deg = (counts[0] + counts[1]) * (1.0 / 16.0) + 1.0
    dis = lax.rsqrt(deg)
    dis_ref[...] = dis
    g_ref[...] = dis * p_ref[...]

  return pl.pallas_call(
      body,
      grid=(n // bn,),
      in_specs=[
          pl.BlockSpec((NC, bn, 16), lambda i: (0, i, 0)),
          pl.BlockSpec((bn, d), lambda i: (i, 0)),
      ],
      out_specs=[
          pl.BlockSpec((bn, 1), lambda i: (i, 0)),
          pl.BlockSpec((bn, d), lambda i: (i, 0)),
      ],
      out_shape=[
          jax.ShapeDtypeStruct((n, 1), jnp.float32),
          jax.ShapeDtypeStruct((n, d), jnp.float32),
      ],
  )(deg16, p1)


def _tc_mid(t, g, dis, b, w, bn):
  n, d = g.shape

  def body(t_ref, g_ref, dis_ref, b_ref, w_ref, o_ref):
    tsum = t_ref[0] + t_ref[1]
    h = jnp.maximum(dis_ref[...] * (tsum + g_ref[...]) + b_ref[...], 0.0)
    o_ref[...] = dis_ref[...] * _dot(h, w_ref[...])

  return pl.pallas_call(
      body,
      grid=(n // bn,),
      in_specs=[
          pl.BlockSpec((NC, bn, d), lambda i: (0, i, 0)),
          pl.BlockSpec((bn, d), lambda i: (i, 0)),
          pl.BlockSpec((bn, 1), lambda i: (i, 0)),
          pl.BlockSpec((1, d), lambda i: (0, 0)),
          pl.BlockSpec((d, d), lambda i: (0, 0)),
      ],
      out_specs=pl.BlockSpec((bn, d), lambda i: (i, 0)),
      out_shape=jax.ShapeDtypeStruct((n, d), jnp.float32),
  )(t, g, dis, b.reshape(1, d), w)


def _tc_fin(t, g, dis, b, bn):
  n, d = g.shape

  def body(t_ref, g_ref, dis_ref, b_ref, o_ref):
    tsum = t_ref[0] + t_ref[1]
    o_ref[...] = dis_ref[...] * (tsum + g_ref[...]) + b_ref[...]

  return pl.pallas_call(
      body,
      grid=(n // bn,),
      in_specs=[
          pl.BlockSpec((NC, bn, d), lambda i: (0, i, 0)),
          pl.BlockSpec((bn, d), lambda i: (i, 0)),
          pl.BlockSpec((bn, 1), lambda i: (i, 0)),
          pl.BlockSpec((1, d), lambda i: (0, 0)),
      ],
      out_specs=pl.BlockSpec((bn, d), lambda i: (i, 0)),
      out_shape=jax.ShapeDtypeStruct((n, d), jnp.float32),
  )(t, g, dis, b.reshape(1, d))


def kernel(x, edge_index, W1, b1, W2, b2, W3, b3):
  n, d = x.shape
  e = edge_index.shape[1]

  k = -(-e // (NC * NS * CH))
  ep = NC * NS * k * CH
  pad = ep - e
  rows = jnp.concatenate(
      [edge_index[0], jnp.zeros((pad,), jnp.int32)]).reshape(NC, NS, k, CH)
  cols = jnp.concatenate(
      [edge_index[1], jnp.full((pad,), n, jnp.int32)]).reshape(NC, NS, k, CH)

  acc_rows = (n // (NS * 8) + 1) * NS * 8

  ones16 = jnp.ones((CH, 16), jnp.float32)
  zeros16 = jnp.zeros((acc_rows, 16), jnp.float32)
  zeros128 = jnp.zeros((acc_rows, d), jnp.float32)

  bn = 1000

  deg16 = _sc_degree(cols, ones16, zeros16, k, acc_rows)
  p1 = _tc_mm(x, W1, bn)
  dis, g1 = _tc_scale0(deg16, p1, bn)

  t1 = _sc_aggregate(g1, rows, cols, zeros128, k, acc_rows)
  g2 = _tc_mid(t1, g1, dis, b1, W2, bn)
  t2 = _sc_aggregate(g2, rows, cols, zeros128, k, acc_rows)
  g3 = _tc_mid(t2, g2, dis, b2, W3, bn)
  t3 = _sc_aggregate(g3, rows, cols, zeros128, k, acc_rows)
  return _tc_fin(t3, g3, dis, b3, bn)
